# baseline (speedup 1.0000x reference)
import functools

import jax
import jax.numpy as jnp
from jax import lax
from jax.experimental import pallas as pl
from jax.experimental.pallas import tpu as pltpu
from jax.experimental.pallas import tpu_sc as plsc

NUM_AGENTS = 8
HIDDEN = 1024
NZ = 2048
BATCH = 16384
ROWS = 1024

NW = 32
STRIPE = 128
NSTRIPES_PER_CORE = (NZ // STRIPE) // 2
HALF_ROWS = (NUM_AGENTS * HIDDEN) // 2
AG_HALF = NUM_AGENTS // 2
CHUNK_ROWS = 128
NCHUNKS = HALF_ROWS // CHUNK_ROWS
CHUNKS_PER_AGENT = HIDDEN // CHUNK_ROWS
NBUF = 4
L = 16
G = STRIPE // L
GH = G // 2


def _sc_scale_body(b_hbm, w_hbm, aidx_hbm, out_hbm,
                   buf0, buf1, buf2, buf3, sums_ref, part_ref, shared_ref,
                   wv_ref, aidxv_ref, scale_ref, sem0, sem1, sem2, sem3):
    cid = lax.axis_index("c")
    sid = lax.axis_index("s")
    stripe = lax.rem(sid, NSTRIPES_PER_CORE)
    half = sid // NSTRIPES_PER_CORE
    col0 = (cid * NSTRIPES_PER_CORE + stripe) * STRIPE
    row0 = half * HALF_ROWS

    pltpu.sync_copy(w_hbm, wv_ref)
    pltpu.sync_copy(aidx_hbm, aidxv_ref)
    aidx_v = aidxv_ref[...]

    bufs = [buf0, buf1, buf2, buf3]
    sems = [sem0, sem1, sem2, sem3]

    def start(c):
        return pltpu.async_copy(
            b_hbm.at[pl.ds(row0 + c * CHUNK_ROWS, CHUNK_ROWS),
                     pl.ds(col0, STRIPE)],
            bufs[c % NBUF], sems[c % NBUF])

    handles = {}
    for c in range(NBUF):
        handles[c] = start(c)

    for c in range(NCHUNKS):
        handles[c].wait()
        a = c // CHUNKS_PER_AGENT
        buf = bufs[c % NBUF]

        def row_body(i, accs, buf=buf):
            r = i * 2
            t = tuple(accs[g] + buf[r, pl.ds(L * g, L)] for g in range(G))
            return tuple(t[g] + buf[r + 1, pl.ds(L * g, L)] for g in range(G))

        accs = lax.fori_loop(
            0, CHUNK_ROWS // 2, row_body,
            tuple(jnp.zeros((L,), jnp.int32) for _ in range(G)))
        for g in range(G):
            if c % CHUNKS_PER_AGENT == 0:
                sums_ref[a, pl.ds(L * g, L)] = accs[g]
            else:
                prev = sums_ref[a, pl.ds(L * g, L)]
                sums_ref[a, pl.ds(L * g, L)] = prev + accs[g]
        if c + NBUF < NCHUNKS:
            handles[c + NBUF] = start(c + NBUF)

    pltpu.sync_copy(sums_ref, shared_ref.at[sid])
    plsc.subcore_barrier()
    partner = lax.rem(sid + NSTRIPES_PER_CORE, 2 * NSTRIPES_PER_CORE)
    pltpu.sync_copy(shared_ref.at[partner], part_ref)

    def _finalize(h):
        for gl in range(GH):
            sl = pl.ds(h * (GH * L) + L * gl, L)
            alpha = jnp.zeros((L,), jnp.float32)
            msel = jnp.zeros((L,), jnp.float32)
            for al in range(AG_HALF):
                relf = jnp.minimum(sums_ref[al, sl], 1).astype(jnp.float32)
                alpha = alpha + relf
                msel = jnp.where(aidx_v == h * AG_HALF + al, relf, msel)
                relp = jnp.minimum(part_ref[al, sl], 1).astype(jnp.float32)
                alpha = alpha + relp
                msel = jnp.where(aidx_v == (1 - h) * AG_HALF + al,
                                 relp, msel)
            weights = jnp.zeros((L,), jnp.float32)
            for k in range(NUM_AGENTS + 1):
                wk = wv_ref[pl.ds(L * k, L)]
                weights = jnp.where(alpha == float(k), wk, weights)
            scale_ref[pl.ds(L * gl, L)] = msel * weights

    @pl.when(half == 0)
    def _half0():
        _finalize(0)

    @pl.when(half == 1)
    def _half1():
        _finalize(1)

    pltpu.sync_copy(scale_ref,
                    out_hbm.at[pl.ds(col0 + half * (GH * L), GH * L)])


_sc_scale = functools.partial(
    pl.kernel,
    out_type=jax.ShapeDtypeStruct((NZ,), jnp.float32),
    mesh=plsc.VectorSubcoreMesh(core_axis_name="c", subcore_axis_name="s"),
    scratch_types=[
        pltpu.VMEM((CHUNK_ROWS, STRIPE), jnp.int32),
        pltpu.VMEM((CHUNK_ROWS, STRIPE), jnp.int32),
        pltpu.VMEM((CHUNK_ROWS, STRIPE), jnp.int32),
        pltpu.VMEM((CHUNK_ROWS, STRIPE), jnp.int32),
        pltpu.VMEM((AG_HALF, STRIPE), jnp.int32),
        pltpu.VMEM((AG_HALF, STRIPE), jnp.int32),
        pltpu.VMEM_SHARED((16, AG_HALF, STRIPE), jnp.int32),
        pltpu.VMEM(((NUM_AGENTS + 1) * L,), jnp.float32),
        pltpu.VMEM((L,), jnp.int32),
        pltpu.VMEM((GH * L,), jnp.float32),
        pltpu.SemaphoreType.DMA,
        pltpu.SemaphoreType.DMA,
        pltpu.SemaphoreType.DMA,
        pltpu.SemaphoreType.DMA,
    ],
)(_sc_scale_body)


def _mul_kernel(z_ref, s_ref, out_ref):
    out_ref[...] = z_ref[...] * s_ref[...]


@functools.partial(jax.jit, static_argnames=())
def kernel(Z_hat, B, w, agent_idx):
    w_b = jnp.broadcast_to(w[:, None], (NUM_AGENTS + 1, L)).reshape(-1)
    aidx_b = jnp.full((L,), agent_idx, jnp.int32)

    scale = _sc_scale(B, w_b, aidx_b).reshape(1, NZ)

    out = pl.pallas_call(
        _mul_kernel,
        grid=(BATCH // ROWS,),
        in_specs=[
            pl.BlockSpec((ROWS, NZ), lambda i: (i, 0)),
            pl.BlockSpec((1, NZ), lambda i: (0, 0)),
        ],
        out_specs=pl.BlockSpec((ROWS, NZ), lambda i: (i, 0)),
        out_shape=jax.ShapeDtypeStruct((BATCH, NZ), jnp.float32),
    )(Z_hat, scale)
    return out
